# baseline (speedup 1.0000x reference)
import functools

import jax
import jax.numpy as jnp
from jax import lax
from jax.experimental import pallas as pl
from jax.experimental.pallas import tpu as pltpu
from jax.experimental.pallas import tpu_sc as plsc

N = 10000
E = 320000
D = 128
H = 64
G = 64

NW = 32
R = 320
NPAD = 10240
C = 2560
BB = 512
GCAP = BB + C + 16
TRASH = GCAP
GBUF = GCAP + 16
ECAP = 640 * BB
NEG = float("-inf")

_SC_PARAMS = pltpu.CompilerParams(
    needs_layout_passes=False, use_tc_tiling_on_sc=False)
_MESH = plsc.VectorSubcoreMesh(core_axis_name="c", subcore_axis_name="s")


@functools.partial(
    pl.kernel,
    mesh=_MESH,
    out_type=(
        jax.ShapeDtypeStruct((NW, ECAP), jnp.int32),
        jax.ShapeDtypeStruct((NW, ECAP), jnp.int32),
        jax.ShapeDtypeStruct((NW, 16), jnp.int32),
    ),
    scratch_types=[
        pltpu.VMEM((C,), jnp.int32),
        pltpu.VMEM((C,), jnp.int32),
        pltpu.VMEM((GBUF,), jnp.int32),
        pltpu.VMEM((GBUF,), jnp.int32),
        pltpu.VMEM((16,), jnp.int32),
    ],
    compiler_params=_SC_PARAMS,
)
def _sc_bin(src_hbm, dst_hbm, elsrc_hbm, eldl_hbm, cnt_hbm,
            srcbuf, dstbuf, gsrc, gdl, cntv):
    wid = lax.axis_index("s") * 2 + lax.axis_index("c")
    lo = wid * R

    def flush(t):
        pltpu.sync_copy(gsrc.at[pl.ds(0, BB)],
                        elsrc_hbm.at[wid, pl.ds(t * BB, BB)])
        pltpu.sync_copy(gdl.at[pl.ds(0, BB)],
                        eldl_hbm.at[wid, pl.ds(t * BB, BB)])

    def drain(st):
        fill, t = st
        flush(t)
        rem = fill - BB
        nmove = (rem + 15) // 16

        def mv(i, carry):
            gsrc[pl.ds(16 * i, 16)] = gsrc[pl.ds(BB + 16 * i, 16)]
            gdl[pl.ds(16 * i, 16)] = gdl[pl.ds(BB + 16 * i, 16)]
            return carry
        lax.fori_loop(0, nmove, mv, 0)
        return rem, t + 1

    def chunk_step(c, st):
        pltpu.sync_copy(src_hbm.at[pl.ds(c * C, C)], srcbuf)
        pltpu.sync_copy(dst_hbm.at[pl.ds(c * C, C)], dstbuf)

        def filt(j, fl):
            d = dstbuf[pl.ds(16 * j, 16)]
            s = srcbuf[pl.ds(16 * j, 16)]
            m = (d >= lo) & (d < lo + R)
            pos = plsc.cumsum(jnp.where(m, 1, 0))
            idx = jnp.where(m, fl + pos - 1, TRASH)
            plsc.store_scatter(gsrc, [idx], s)
            plsc.store_scatter(gdl, [idx], jnp.where(m, d - lo, R))
            return fl + pos[15]
        fill = lax.fori_loop(0, C // 16, filt, st[0])
        return lax.while_loop(lambda s2: s2[0] >= BB, drain, (fill, st[1]))

    fill, t = lax.fori_loop(0, E // C, chunk_step, (0, 0))

    pad_src = jnp.zeros((16,), dtype=jnp.int32)
    pad_dl = jnp.full((16,), R, dtype=jnp.int32)
    for p in range(BB // 16):
        gsrc[pl.ds(fill + 16 * p, 16)] = pad_src
        gdl[pl.ds(fill + 16 * p, 16)] = pad_dl
    flush(t)

    T = t + 1
    for p in range(BB // 16):
        gsrc[pl.ds(16 * p, 16)] = pad_src
        gdl[pl.ds(16 * p, 16)] = pad_dl
    M = 2 * ((T + 1) // 2) + 4

    def padflush(t2, carry):
        flush(t2)
        return carry
    lax.fori_loop(T, M, padflush, 0)

    cntv[pl.ds(0, 16)] = jnp.full((16,), T, dtype=jnp.int32)
    pltpu.sync_copy(cntv, cnt_hbm.at[wid])


def _make_sc_layer(F, CGf):
    fvecs = F // 16

    @functools.partial(
        pl.kernel,
        mesh=_MESH,
        out_type=jax.ShapeDtypeStruct((NPAD, F), jnp.float32),
        scratch_types=[
            pltpu.VMEM((CGf,), jnp.int32),
            pltpu.VMEM((CGf,), jnp.int32),
            pltpu.VMEM((CGf,), jnp.int32),
            pltpu.VMEM((CGf,), jnp.int32),
            pltpu.VMEM((CGf, F), jnp.float32),
            pltpu.VMEM((CGf, F), jnp.float32),
            pltpu.VMEM((R + 1, F), jnp.float32),
            pltpu.VMEM((16,), jnp.int32),
            pltpu.SemaphoreType.DMA,
            pltpu.SemaphoreType.DMA,
        ],
        compiler_params=_SC_PARAMS,
    )
    def sc_layer(elsrc_hbm, eldl_hbm, cnt_hbm, x_hbm, out_hbm,
                 bsrc0, bdl0, bsrc1, bdl1, rows0, rows1, acc, cntv,
                 sem0, sem1):
        wid = lax.axis_index("s") * 2 + lax.axis_index("c")

        pltpu.sync_copy(cnt_hbm.at[wid], cntv)
        tv = cntv[pl.ds(0, 16)]
        nb = tv[0] * (BB // CGf)
        nbo = (nb + 1) // 2

        neg = jnp.full((16,), NEG, dtype=jnp.float32)

        def init_row(r, carry):
            for f in range(fvecs):
                acc[r, pl.ds(16 * f, 16)] = neg
            return carry
        lax.fori_loop(0, R + 1, init_row, 0)

        def load_idx(b, dst_src, dst_dl):
            pltpu.sync_copy(elsrc_hbm.at[wid, pl.ds(b * CGf, CGf)], dst_src)
            pltpu.sync_copy(eldl_hbm.at[wid, pl.ds(b * CGf, CGf)], dst_dl)

        def process(rows, bdl):
            def pg(eg, carry):
                dls = bdl[pl.ds(16 * eg, 16)]
                for k in range(16):
                    dl = dls[k]
                    for f in range(fvecs):
                        sl = pl.ds(16 * f, 16)
                        acc[dl, sl] = jnp.maximum(acc[dl, sl],
                                                  rows[16 * eg + k, sl])
                return carry
            lax.fori_loop(0, CGf // 16, pg, 0)

        load_idx(0, bsrc0, bdl0)
        load_idx(1, bsrc1, bdl1)
        pltpu.async_copy(x_hbm.at[bsrc0], rows0, sem0)

        def pair(o, carry):
            pltpu.async_copy(x_hbm.at[bsrc1], rows1, sem1)
            pltpu.make_async_copy(x_hbm.at[bsrc0], rows0, sem0).wait()
            process(rows0, bdl0)
            load_idx(2 * o + 2, bsrc0, bdl0)
            pltpu.async_copy(x_hbm.at[bsrc0], rows0, sem0)
            pltpu.make_async_copy(x_hbm.at[bsrc1], rows1, sem1).wait()
            process(rows1, bdl1)
            load_idx(2 * o + 3, bsrc1, bdl1)
            return carry
        lax.fori_loop(0, nbo, pair, 0)
        pltpu.make_async_copy(x_hbm.at[bsrc0], rows0, sem0).wait()

        def fix_row(r, carry):
            for f in range(fvecs):
                sl = pl.ds(16 * f, 16)
                v = acc[r, sl]
                acc[r, sl] = jnp.where(v == NEG, 0.0, v)
            return carry
        lax.fori_loop(0, R, fix_row, 0)
        pltpu.sync_copy(acc.at[pl.ds(0, R)], out_hbm.at[pl.ds(wid * R, R)])

    return sc_layer


_sc_layer_d = _make_sc_layer(D, 256)
_sc_layer_h = _make_sc_layer(H, 512)


def _tc_layer(agg, h, Wl, Wr, b):
    BN = 512
    npad, fa = agg.shape
    fh = h.shape[1]
    b2 = b.reshape(1, H)

    def body(agg_ref, h_ref, wl_ref, wr_ref, b_ref, o_ref):
        acc = jnp.dot(agg_ref[...], wl_ref[...],
                      preferred_element_type=jnp.float32)
        acc += jnp.dot(h_ref[...], wr_ref[...],
                       preferred_element_type=jnp.float32)
        o_ref[...] = jnp.maximum(acc + b_ref[...], 0.0)

    return pl.pallas_call(
        body,
        grid=(npad // BN,),
        in_specs=[
            pl.BlockSpec((BN, fa), lambda i: (i, 0)),
            pl.BlockSpec((BN, fh), lambda i: (i, 0)),
            pl.BlockSpec((fa, H), lambda i: (0, 0)),
            pl.BlockSpec((fh, H), lambda i: (0, 0)),
            pl.BlockSpec((1, H), lambda i: (0, 0)),
        ],
        out_specs=pl.BlockSpec((BN, H), lambda i: (i, 0)),
        out_shape=jax.ShapeDtypeStruct((npad, H), jnp.float32),
    )(agg, h, Wl, Wr, b2)


def _tc_pool_mlp(h3, batchp, A1, ab1, A2, ab2):
    BN = 512
    npad = h3.shape[0]
    ys = A2.shape[1]
    a1b = ab1.reshape(1, A1.shape[1])
    a2b = ab2.reshape(1, ys)

    def body(h_ref, b_ref, a1_ref, ab1_ref, a2_ref, ab2_ref, o_ref, acc_ref):
        i = pl.program_id(0)

        @pl.when(i == 0)
        def _():
            acc_ref[...] = jnp.full_like(acc_ref, NEG)

        hb = h_ref[...]
        ids = b_ref[...]
        parts = [
            jnp.max(jnp.where(ids == g, hb, NEG), axis=0, keepdims=True)
            for g in range(G)
        ]
        acc_ref[...] = jnp.maximum(acc_ref[...], jnp.concatenate(parts, 0))

        @pl.when(i == pl.num_programs(0) - 1)
        def _():
            pooled = acc_ref[...]
            pooled = jnp.where(pooled == NEG, 0.0, pooled)
            t = jnp.dot(pooled, a1_ref[...], preferred_element_type=jnp.float32)
            t = jnp.maximum(t + ab1_ref[...], 0.0)
            o_ref[...] = jnp.dot(t, a2_ref[...],
                                 preferred_element_type=jnp.float32) + ab2_ref[...]

    return pl.pallas_call(
        body,
        grid=(npad // BN,),
        in_specs=[
            pl.BlockSpec((BN, H), lambda i: (i, 0)),
            pl.BlockSpec((BN, 1), lambda i: (i, 0)),
            pl.BlockSpec(A1.shape, lambda i: (0, 0)),
            pl.BlockSpec((1, A1.shape[1]), lambda i: (0, 0)),
            pl.BlockSpec(A2.shape, lambda i: (0, 0)),
            pl.BlockSpec((1, ys), lambda i: (0, 0)),
        ],
        out_specs=pl.BlockSpec((G, ys), lambda i: (0, 0)),
        out_shape=jax.ShapeDtypeStruct((G, ys), jnp.float32),
        scratch_shapes=[pltpu.VMEM((G, H), jnp.float32)],
    )(h3, batchp, A1, a1b, A2, a2b)


def kernel(x, edge_index, batch, W1l, W1r, b1, W2l, W2r, b2, W3l, W3r, b3,
           A1, ab1, A2, ab2):
    src = edge_index[0]
    dst = edge_index[1]
    x_p = jnp.zeros((NPAD, D), jnp.float32).at[:N].set(x)
    batchp = jnp.concatenate(
        [batch, jnp.full((NPAD - N,), G, jnp.int32)]).reshape(NPAD, 1)

    elsrc, eldl, cnt = _sc_bin(src, dst)
    agg1 = _sc_layer_d(elsrc, eldl, cnt, x_p)
    h1 = _tc_layer(agg1, x_p, W1l, W1r, b1)
    agg2 = _sc_layer_h(elsrc, eldl, cnt, h1)
    h2 = _tc_layer(agg2, h1, W2l, W2r, b2)
    agg3 = _sc_layer_h(elsrc, eldl, cnt, h2)
    h3 = _tc_layer(agg3, h2, W3l, W3r, b3)
    return _tc_pool_mlp(h3, batchp, A1, ab1, A2, ab2)


# speedup vs baseline: 1.0363x; 1.0363x over previous
import functools

import jax
import jax.numpy as jnp
from jax import lax
from jax.experimental import pallas as pl
from jax.experimental.pallas import tpu as pltpu
from jax.experimental.pallas import tpu_sc as plsc

N = 10000
E = 320000
D = 128
H = 64
G = 64

NW = 32
R = 320
NPAD = 10240
C = 2560
BB = 512
GCAP = BB + C + 16
TRASH = GCAP
GBUF = GCAP + 16
ECAP = 640 * BB
NEG = float("-inf")

_SC_PARAMS = pltpu.CompilerParams(
    needs_layout_passes=False, use_tc_tiling_on_sc=False)
_MESH = plsc.VectorSubcoreMesh(core_axis_name="c", subcore_axis_name="s")


@functools.partial(
    pl.kernel,
    mesh=_MESH,
    out_type=(
        jax.ShapeDtypeStruct((NW, ECAP), jnp.int32),
        jax.ShapeDtypeStruct((NW, ECAP), jnp.int32),
        jax.ShapeDtypeStruct((NW, 16), jnp.int32),
    ),
    scratch_types=[
        pltpu.VMEM((C,), jnp.int32),
        pltpu.VMEM((C,), jnp.int32),
        pltpu.VMEM((GBUF,), jnp.int32),
        pltpu.VMEM((GBUF,), jnp.int32),
        pltpu.VMEM((16,), jnp.int32),
    ],
    compiler_params=_SC_PARAMS,
)
def _sc_bin(src_hbm, dst_hbm, elsrc_hbm, eldl_hbm, cnt_hbm,
            srcbuf, dstbuf, gsrc, gdl, cntv):
    wid = lax.axis_index("s") * 2 + lax.axis_index("c")
    lo = wid * R

    def flush(t):
        pltpu.sync_copy(gsrc.at[pl.ds(0, BB)],
                        elsrc_hbm.at[wid, pl.ds(t * BB, BB)])
        pltpu.sync_copy(gdl.at[pl.ds(0, BB)],
                        eldl_hbm.at[wid, pl.ds(t * BB, BB)])

    def drain(st):
        fill, t = st
        flush(t)
        rem = fill - BB
        nmove = (rem + 15) // 16

        def mv(i, carry):
            gsrc[pl.ds(16 * i, 16)] = gsrc[pl.ds(BB + 16 * i, 16)]
            gdl[pl.ds(16 * i, 16)] = gdl[pl.ds(BB + 16 * i, 16)]
            return carry
        lax.fori_loop(0, nmove, mv, 0)
        return rem, t + 1

    def chunk_step(c, st):
        pltpu.sync_copy(src_hbm.at[pl.ds(c * C, C)], srcbuf)
        pltpu.sync_copy(dst_hbm.at[pl.ds(c * C, C)], dstbuf)

        def filt(j, fl):
            d = dstbuf[pl.ds(16 * j, 16)]
            s = srcbuf[pl.ds(16 * j, 16)]
            m = (d >= lo) & (d < lo + R)
            pos = plsc.cumsum(jnp.where(m, 1, 0))
            idx = jnp.where(m, fl + pos - 1, TRASH)
            plsc.store_scatter(gsrc, [idx], s)
            plsc.store_scatter(gdl, [idx], jnp.where(m, d - lo, R))
            return fl + pos[15]
        fill = lax.fori_loop(0, C // 16, filt, st[0])
        return lax.while_loop(lambda s2: s2[0] >= BB, drain, (fill, st[1]))

    fill, t = lax.fori_loop(0, E // C, chunk_step, (0, 0))

    pad_src = jnp.zeros((16,), dtype=jnp.int32)
    pad_dl = jnp.full((16,), R, dtype=jnp.int32)
    for p in range(BB // 16):
        gsrc[pl.ds(fill + 16 * p, 16)] = pad_src
        gdl[pl.ds(fill + 16 * p, 16)] = pad_dl
    flush(t)

    T = t + 1
    for p in range(BB // 16):
        gsrc[pl.ds(16 * p, 16)] = pad_src
        gdl[pl.ds(16 * p, 16)] = pad_dl
    M = 2 * ((T + 1) // 2) + 4

    def padflush(t2, carry):
        flush(t2)
        return carry
    lax.fori_loop(T, M, padflush, 0)

    cntv[pl.ds(0, 16)] = jnp.full((16,), T, dtype=jnp.int32)
    pltpu.sync_copy(cntv, cnt_hbm.at[wid])


def _make_sc_layer(F, CGf):
    fvecs = F // 16

    @functools.partial(
        pl.kernel,
        mesh=_MESH,
        out_type=jax.ShapeDtypeStruct((NPAD, F), jnp.float32),
        scratch_types=[
            pltpu.VMEM((CGf,), jnp.int32),
            pltpu.VMEM((CGf,), jnp.int32),
            pltpu.VMEM((CGf,), jnp.int32),
            pltpu.VMEM((CGf,), jnp.int32),
            pltpu.VMEM((CGf, F), jnp.float32),
            pltpu.VMEM((CGf, F), jnp.float32),
            pltpu.VMEM((R + 1, F), jnp.float32),
            pltpu.VMEM((R + 1, F), jnp.float32),
            pltpu.VMEM((16,), jnp.int32),
            pltpu.SemaphoreType.DMA,
            pltpu.SemaphoreType.DMA,
        ],
        compiler_params=_SC_PARAMS,
    )
    def sc_layer(elsrc_hbm, eldl_hbm, cnt_hbm, x_hbm, out_hbm,
                 bsrc0, bdl0, bsrc1, bdl1, rows0, rows1, acc, accb, cntv,
                 sem0, sem1):
        wid = lax.axis_index("s") * 2 + lax.axis_index("c")

        pltpu.sync_copy(cnt_hbm.at[wid], cntv)
        tv = cntv[pl.ds(0, 16)]
        nb = tv[0] * (BB // CGf)
        nbo = (nb + 1) // 2

        neg = jnp.full((16,), NEG, dtype=jnp.float32)

        def init_row(r, carry):
            for f in range(fvecs):
                acc[r, pl.ds(16 * f, 16)] = neg
                accb[r, pl.ds(16 * f, 16)] = neg
            return carry
        lax.fori_loop(0, R + 1, init_row, 0)

        def load_idx(b, dst_src, dst_dl):
            pltpu.sync_copy(elsrc_hbm.at[wid, pl.ds(b * CGf, CGf)], dst_src)
            pltpu.sync_copy(eldl_hbm.at[wid, pl.ds(b * CGf, CGf)], dst_dl)

        def process(rows, bdl):
            def pg(eg, carry):
                dls = bdl[pl.ds(16 * eg, 16)]
                for k in range(16):
                    dl = dls[k]
                    a = acc if k % 2 == 0 else accb
                    for f in range(fvecs):
                        sl = pl.ds(16 * f, 16)
                        a[dl, sl] = jnp.maximum(a[dl, sl],
                                                rows[16 * eg + k, sl])
                return carry
            lax.fori_loop(0, CGf // 16, pg, 0)

        load_idx(0, bsrc0, bdl0)
        load_idx(1, bsrc1, bdl1)
        pltpu.async_copy(x_hbm.at[bsrc0], rows0, sem0)

        def pair(o, carry):
            pltpu.async_copy(x_hbm.at[bsrc1], rows1, sem1)
            pltpu.make_async_copy(x_hbm.at[bsrc0], rows0, sem0).wait()
            process(rows0, bdl0)
            load_idx(2 * o + 2, bsrc0, bdl0)
            pltpu.async_copy(x_hbm.at[bsrc0], rows0, sem0)
            pltpu.make_async_copy(x_hbm.at[bsrc1], rows1, sem1).wait()
            process(rows1, bdl1)
            load_idx(2 * o + 3, bsrc1, bdl1)
            return carry
        lax.fori_loop(0, nbo, pair, 0)
        pltpu.make_async_copy(x_hbm.at[bsrc0], rows0, sem0).wait()

        def fix_row(r, carry):
            for f in range(fvecs):
                sl = pl.ds(16 * f, 16)
                v = jnp.maximum(acc[r, sl], accb[r, sl])
                acc[r, sl] = jnp.where(v == NEG, 0.0, v)
            return carry
        lax.fori_loop(0, R, fix_row, 0)
        pltpu.sync_copy(acc.at[pl.ds(0, R)], out_hbm.at[pl.ds(wid * R, R)])

    return sc_layer


_sc_layer_d = _make_sc_layer(D, 128)
_sc_layer_h = _make_sc_layer(H, 512)


def _tc_layer(agg, h, Wl, Wr, b):
    BN = 512
    npad, fa = agg.shape
    fh = h.shape[1]
    b2 = b.reshape(1, H)

    def body(agg_ref, h_ref, wl_ref, wr_ref, b_ref, o_ref):
        acc = jnp.dot(agg_ref[...], wl_ref[...],
                      preferred_element_type=jnp.float32)
        acc += jnp.dot(h_ref[...], wr_ref[...],
                       preferred_element_type=jnp.float32)
        o_ref[...] = jnp.maximum(acc + b_ref[...], 0.0)

    return pl.pallas_call(
        body,
        grid=(npad // BN,),
        in_specs=[
            pl.BlockSpec((BN, fa), lambda i: (i, 0)),
            pl.BlockSpec((BN, fh), lambda i: (i, 0)),
            pl.BlockSpec((fa, H), lambda i: (0, 0)),
            pl.BlockSpec((fh, H), lambda i: (0, 0)),
            pl.BlockSpec((1, H), lambda i: (0, 0)),
        ],
        out_specs=pl.BlockSpec((BN, H), lambda i: (i, 0)),
        out_shape=jax.ShapeDtypeStruct((npad, H), jnp.float32),
    )(agg, h, Wl, Wr, b2)


def _tc_pool_mlp(h3, batchp, A1, ab1, A2, ab2):
    BN = 512
    npad = h3.shape[0]
    ys = A2.shape[1]
    a1b = ab1.reshape(1, A1.shape[1])
    a2b = ab2.reshape(1, ys)

    def body(h_ref, b_ref, a1_ref, ab1_ref, a2_ref, ab2_ref, o_ref, acc_ref):
        i = pl.program_id(0)

        @pl.when(i == 0)
        def _():
            acc_ref[...] = jnp.full_like(acc_ref, NEG)

        hb = h_ref[...]
        ids = b_ref[...]
        parts = [
            jnp.max(jnp.where(ids == g, hb, NEG), axis=0, keepdims=True)
            for g in range(G)
        ]
        acc_ref[...] = jnp.maximum(acc_ref[...], jnp.concatenate(parts, 0))

        @pl.when(i == pl.num_programs(0) - 1)
        def _():
            pooled = acc_ref[...]
            pooled = jnp.where(pooled == NEG, 0.0, pooled)
            t = jnp.dot(pooled, a1_ref[...], preferred_element_type=jnp.float32)
            t = jnp.maximum(t + ab1_ref[...], 0.0)
            o_ref[...] = jnp.dot(t, a2_ref[...],
                                 preferred_element_type=jnp.float32) + ab2_ref[...]

    return pl.pallas_call(
        body,
        grid=(npad // BN,),
        in_specs=[
            pl.BlockSpec((BN, H), lambda i: (i, 0)),
            pl.BlockSpec((BN, 1), lambda i: (i, 0)),
            pl.BlockSpec(A1.shape, lambda i: (0, 0)),
            pl.BlockSpec((1, A1.shape[1]), lambda i: (0, 0)),
            pl.BlockSpec(A2.shape, lambda i: (0, 0)),
            pl.BlockSpec((1, ys), lambda i: (0, 0)),
        ],
        out_specs=pl.BlockSpec((G, ys), lambda i: (0, 0)),
        out_shape=jax.ShapeDtypeStruct((G, ys), jnp.float32),
        scratch_shapes=[pltpu.VMEM((G, H), jnp.float32)],
    )(h3, batchp, A1, a1b, A2, a2b)


def kernel(x, edge_index, batch, W1l, W1r, b1, W2l, W2r, b2, W3l, W3r, b3,
           A1, ab1, A2, ab2):
    src = edge_index[0]
    dst = edge_index[1]
    x_p = jnp.zeros((NPAD, D), jnp.float32).at[:N].set(x)
    batchp = jnp.concatenate(
        [batch, jnp.full((NPAD - N,), G, jnp.int32)]).reshape(NPAD, 1)

    elsrc, eldl, cnt = _sc_bin(src, dst)
    agg1 = _sc_layer_d(elsrc, eldl, cnt, x_p)
    h1 = _tc_layer(agg1, x_p, W1l, W1r, b1)
    agg2 = _sc_layer_h(elsrc, eldl, cnt, h1)
    h2 = _tc_layer(agg2, h1, W2l, W2r, b2)
    agg3 = _sc_layer_h(elsrc, eldl, cnt, h2)
    h3 = _tc_layer(agg3, h2, W3l, W3r, b3)
    return _tc_pool_mlp(h3, batchp, A1, ab1, A2, ab2)
